# baseline (speedup 1.0000x reference)
import functools

import jax
import jax.numpy as jnp
from jax import lax
from jax.experimental import pallas as pl
from jax.experimental.pallas import tpu as pltpu
from jax.experimental.pallas import tpu_sc as plsc

D = 128
V0, V1, V2 = 5, 6, 2
TROWS = 64


def _combine_body(w0_ref, w1_ref, w2_ref, out_ref):
    code = lax.broadcasted_iota(jnp.int32, (TROWS, 1), 0)
    i0 = code // (V1 * V2)
    i1 = (code // V2) % V1
    i2 = code % V2
    acc = jnp.zeros((TROWS, D), jnp.float32)
    for v in range(V0):
        acc = acc + jnp.where(i0 == v, w0_ref[v : v + 1, :], 0.0)
    for v in range(V1):
        acc = acc + jnp.where(i1 == v, w1_ref[v : v + 1, :], 0.0)
    for v in range(V2):
        acc = acc + jnp.where(i2 == v, w2_ref[v : v + 1, :], 0.0)
    out_ref[...] = acc


def _combine_tables(W0, W1, W2):
    return pl.pallas_call(
        _combine_body,
        out_shape=jax.ShapeDtypeStruct((TROWS, D), jnp.float32),
    )(W0, W1, W2)


def _make_sc_lookup(E):
    NW = 32
    per_w = E // NW
    CH = 80
    NCH = per_w // CH
    XB = 5
    NBUF = 6
    assert per_w * NW == E and NCH * CH == per_w
    assert CH % 16 == 0 and CH <= 128 and (CH * 3) % 8 == 0
    assert NCH % XB == 0 and NCH > NBUF

    mesh = plsc.VectorSubcoreMesh(core_axis_name="c", subcore_axis_name="s")

    @functools.partial(
        pl.kernel,
        out_type=jax.ShapeDtypeStruct((E, D), jnp.float32),
        mesh=mesh,
        scratch_types=[
            pltpu.VMEM_SHARED((TROWS, D), jnp.float32),
            pltpu.VMEM((2 * XB * CH * 3,), jnp.int32),
            pltpu.VMEM((NBUF * CH,), jnp.int32),
            pltpu.VMEM((NBUF * CH, D), jnp.float32),
            pltpu.SemaphoreType.DMA,
            pltpu.SemaphoreType.DMA,
            pltpu.SemaphoreType.DMA,
        ],
        compiler_params=pltpu.CompilerParams(needs_layout_passes=False),
    )
    def lookup(tcomb_hbm, x_hbm, out_hbm, tcv, xv, codev, rows, xsem, gsem, osem):
        cid = lax.axis_index("c")
        sid = lax.axis_index("s")
        wid = sid * 2 + cid
        base = wid * per_w
        lane = lax.iota(jnp.int32, 16)
        lane3 = lane * 3
        XW = XB * CH * 3

        @pl.when(sid == 0)
        def _():
            pltpu.sync_copy(tcomb_hbm, tcv)

        plsc.subcore_barrier()
        pltpu.async_copy(x_hbm.at[pl.ds(base * 3, XW)], xv.at[pl.ds(0, XW)], xsem)

        def wait_gather():
            pltpu.make_async_copy(
                tcv.at[codev.at[pl.ds(0, CH)]], rows.at[pl.ds(0, CH)], gsem
            ).wait()

        def wait_scatter():
            pltpu.make_async_copy(
                rows.at[pl.ds(0, CH)], out_hbm.at[pl.ds(0, CH)], osem
            ).wait()

        def chunk(m, carry):
            buf = lax.rem(m, NBUF)
            xbuf = lax.rem(m // XB, 2)
            eb = base + m * CH

            @pl.when(lax.rem(m, XB) == 0)
            def _():
                pltpu.make_async_copy(
                    x_hbm.at[pl.ds(0, XW)], xv.at[pl.ds(0, XW)], xsem
                ).wait()

                @pl.when(m + XB < NCH)
                def _():
                    pltpu.async_copy(
                        x_hbm.at[pl.ds((eb + XB * CH) * 3, XW)],
                        xv.at[pl.ds((1 - xbuf) * XW, XW)],
                        xsem,
                    )

            @pl.when(m >= NBUF)
            def _():
                wait_scatter()

            xoff = xbuf * XW + lax.rem(m, XB) * (CH * 3)
            for t in range(CH // 16):
                o = xoff + t * 48
                a = plsc.load_gather(xv, [lane3 + o])
                b = plsc.load_gather(xv, [lane3 + (o + 1)])
                c = plsc.load_gather(xv, [lane3 + (o + 2)])
                a = lax.min(lax.max(a, 0), V0 - 1)
                b = lax.min(lax.max(b, 0), V1 - 1)
                c = lax.min(lax.max(c, 0), V2 - 1)
                codev[pl.ds(buf * CH + t * 16, 16)] = (
                    a * (V1 * V2) + b * V2 + c
                )

            pltpu.async_copy(
                tcv.at[codev.at[pl.ds(buf * CH, CH)]],
                rows.at[pl.ds(buf * CH, CH)],
                gsem,
            )

            @pl.when(m >= 1)
            def _():
                wait_gather()
                prev = lax.rem(m - 1, NBUF)
                pltpu.async_copy(
                    rows.at[pl.ds(prev * CH, CH)],
                    out_hbm.at[pl.ds(eb - CH, CH)],
                    osem,
                )

            return carry

        lax.fori_loop(0, NCH, chunk, 0)

        wait_gather()
        lastbuf = (NCH - 1) % NBUF
        pltpu.async_copy(
            rows.at[pl.ds(lastbuf * CH, CH)],
            out_hbm.at[pl.ds(base + (NCH - 1) * CH, CH)],
            osem,
        )
        for _ in range(NBUF):
            wait_scatter()

    return lookup


def kernel(x, W0, W1, W2):
    E = x.shape[0]
    tcomb = _combine_tables(W0, W1, W2)
    xflat = x.astype(jnp.int32).reshape(-1)
    return _make_sc_lookup(E)(tcomb, xflat)


# speedup vs baseline: 1.0006x; 1.0006x over previous
import functools

import jax
import jax.numpy as jnp
from jax import lax
from jax.experimental import pallas as pl
from jax.experimental.pallas import tpu as pltpu
from jax.experimental.pallas import tpu_sc as plsc

D = 128
V0, V1, V2 = 5, 6, 2
TROWS = 64


def _combine_body(w0_ref, w1_ref, w2_ref, out_ref):
    code = lax.broadcasted_iota(jnp.int32, (TROWS, 1), 0)
    i0 = code // (V1 * V2)
    i1 = (code // V2) % V1
    i2 = code % V2
    acc = jnp.zeros((TROWS, D), jnp.float32)
    for v in range(V0):
        acc = acc + jnp.where(i0 == v, w0_ref[v : v + 1, :], 0.0)
    for v in range(V1):
        acc = acc + jnp.where(i1 == v, w1_ref[v : v + 1, :], 0.0)
    for v in range(V2):
        acc = acc + jnp.where(i2 == v, w2_ref[v : v + 1, :], 0.0)
    out_ref[...] = acc


def _combine_tables(W0, W1, W2):
    return pl.pallas_call(
        _combine_body,
        out_shape=jax.ShapeDtypeStruct((TROWS, D), jnp.float32),
    )(W0, W1, W2)


def _make_sc_lookup(E):
    NW = 32
    per_w = E // NW
    CH = 80
    NCH = per_w // CH
    XB = 5
    NBUF = 10
    assert per_w * NW == E and NCH * CH == per_w
    assert CH % 16 == 0 and CH <= 128 and (CH * 3) % 8 == 0
    assert NCH % XB == 0 and NCH > NBUF

    mesh = plsc.VectorSubcoreMesh(core_axis_name="c", subcore_axis_name="s")

    @functools.partial(
        pl.kernel,
        out_type=jax.ShapeDtypeStruct((E, D), jnp.float32),
        mesh=mesh,
        scratch_types=[
            pltpu.VMEM_SHARED((TROWS, D), jnp.float32),
            pltpu.VMEM((2 * XB * CH * 3,), jnp.int32),
            pltpu.VMEM((NBUF * CH,), jnp.int32),
            pltpu.VMEM((NBUF * CH, D), jnp.float32),
            pltpu.SemaphoreType.DMA,
            pltpu.SemaphoreType.DMA,
            pltpu.SemaphoreType.DMA,
        ],
        compiler_params=pltpu.CompilerParams(needs_layout_passes=False),
    )
    def lookup(tcomb_hbm, x_hbm, out_hbm, tcv, xv, codev, rows, xsem, gsem, osem):
        cid = lax.axis_index("c")
        sid = lax.axis_index("s")
        wid = sid * 2 + cid
        base = wid * per_w
        lane = lax.iota(jnp.int32, 16)
        lane3 = lane * 3
        XW = XB * CH * 3

        @pl.when(sid == 0)
        def _():
            pltpu.sync_copy(tcomb_hbm, tcv)

        plsc.subcore_barrier()
        pltpu.async_copy(x_hbm.at[pl.ds(base * 3, XW)], xv.at[pl.ds(0, XW)], xsem)

        def wait_gather():
            pltpu.make_async_copy(
                tcv.at[codev.at[pl.ds(0, CH)]], rows.at[pl.ds(0, CH)], gsem
            ).wait()

        def wait_scatter():
            pltpu.make_async_copy(
                rows.at[pl.ds(0, CH)], out_hbm.at[pl.ds(0, CH)], osem
            ).wait()

        def chunk(m, carry):
            buf = lax.rem(m, NBUF)
            xbuf = lax.rem(m // XB, 2)
            eb = base + m * CH

            @pl.when(lax.rem(m, XB) == 0)
            def _():
                pltpu.make_async_copy(
                    x_hbm.at[pl.ds(0, XW)], xv.at[pl.ds(0, XW)], xsem
                ).wait()

                @pl.when(m + XB < NCH)
                def _():
                    pltpu.async_copy(
                        x_hbm.at[pl.ds((eb + XB * CH) * 3, XW)],
                        xv.at[pl.ds((1 - xbuf) * XW, XW)],
                        xsem,
                    )

            @pl.when(m >= NBUF)
            def _():
                wait_scatter()

            xoff = xbuf * XW + lax.rem(m, XB) * (CH * 3)
            for t in range(CH // 16):
                o = xoff + t * 48
                a = plsc.load_gather(xv, [lane3 + o])
                b = plsc.load_gather(xv, [lane3 + (o + 1)])
                c = plsc.load_gather(xv, [lane3 + (o + 2)])
                a = lax.min(lax.max(a, 0), V0 - 1)
                b = lax.min(lax.max(b, 0), V1 - 1)
                c = lax.min(lax.max(c, 0), V2 - 1)
                codev[pl.ds(buf * CH + t * 16, 16)] = (
                    a * (V1 * V2) + b * V2 + c
                )

            pltpu.async_copy(
                tcv.at[codev.at[pl.ds(buf * CH, CH)]],
                rows.at[pl.ds(buf * CH, CH)],
                gsem,
            )

            @pl.when(m >= 1)
            def _():
                wait_gather()
                prev = lax.rem(m - 1, NBUF)
                pltpu.async_copy(
                    rows.at[pl.ds(prev * CH, CH)],
                    out_hbm.at[pl.ds(eb - CH, CH)],
                    osem,
                )

            return carry

        lax.fori_loop(0, NCH, chunk, 0)

        wait_gather()
        lastbuf = (NCH - 1) % NBUF
        pltpu.async_copy(
            rows.at[pl.ds(lastbuf * CH, CH)],
            out_hbm.at[pl.ds(base + (NCH - 1) * CH, CH)],
            osem,
        )
        for _ in range(NBUF):
            wait_scatter()

    return lookup


def kernel(x, W0, W1, W2):
    E = x.shape[0]
    tcomb = _combine_tables(W0, W1, W2)
    xflat = x.astype(jnp.int32).reshape(-1)
    return _make_sc_lookup(E)(tcomb, xflat)


# speedup vs baseline: 1.0520x; 1.0514x over previous
import functools

import jax
import jax.numpy as jnp
from jax import lax
from jax.experimental import pallas as pl
from jax.experimental.pallas import tpu as pltpu
from jax.experimental.pallas import tpu_sc as plsc

D = 128
V0, V1, V2 = 5, 6, 2
TROWS = 64


def _combine_body(w0_ref, w1_ref, w2_ref, out_ref):
    code = lax.broadcasted_iota(jnp.int32, (TROWS, 1), 0)
    i0 = code // (V1 * V2)
    i1 = (code // V2) % V1
    i2 = code % V2
    acc = jnp.zeros((TROWS, D), jnp.float32)
    for v in range(V0):
        acc = acc + jnp.where(i0 == v, w0_ref[v : v + 1, :], 0.0)
    for v in range(V1):
        acc = acc + jnp.where(i1 == v, w1_ref[v : v + 1, :], 0.0)
    for v in range(V2):
        acc = acc + jnp.where(i2 == v, w2_ref[v : v + 1, :], 0.0)
    out_ref[...] = acc


def _combine_tables(W0, W1, W2):
    return pl.pallas_call(
        _combine_body,
        out_shape=jax.ShapeDtypeStruct((TROWS, D), jnp.float32),
    )(W0, W1, W2)


def _make_sc_lookup(E):
    NW = 32
    per_w = E // NW
    CH = 80
    NCH = per_w // CH
    NBUF = 6
    assert per_w * NW == E and NCH * CH == per_w
    assert CH % 16 == 0 and CH <= 128 and CH % 8 == 0 and NCH > NBUF

    mesh = plsc.VectorSubcoreMesh(core_axis_name="c", subcore_axis_name="s")

    @functools.partial(
        pl.kernel,
        out_type=jax.ShapeDtypeStruct((E, D), jnp.float32),
        mesh=mesh,
        scratch_types=[
            pltpu.VMEM_SHARED((TROWS, D), jnp.float32),
            pltpu.VMEM((2 * CH, 3), jnp.int32),
            pltpu.VMEM((NBUF * CH,), jnp.int32),
            pltpu.VMEM((NBUF * CH, D), jnp.float32),
            pltpu.SemaphoreType.DMA,
            pltpu.SemaphoreType.DMA,
            pltpu.SemaphoreType.DMA,
        ],
        compiler_params=pltpu.CompilerParams(needs_layout_passes=False),
    )
    def lookup(tcomb_hbm, x2_hbm, out_hbm, tcv, xv, codev, rows, xsem, gsem, osem):
        cid = lax.axis_index("c")
        sid = lax.axis_index("s")
        wid = sid * 2 + cid
        base = wid * per_w
        lane = lax.iota(jnp.int32, 16)
        @pl.when(sid == 0)
        def _():
            pltpu.sync_copy(tcomb_hbm, tcv)

        plsc.subcore_barrier()
        pltpu.async_copy(
            x2_hbm.at[pl.ds(base, CH), :], xv.at[pl.ds(0, CH), :], xsem
        )

        def wait_gather():
            pltpu.make_async_copy(
                tcv.at[codev.at[pl.ds(0, CH)]], rows.at[pl.ds(0, CH)], gsem
            ).wait()

        def wait_scatter():
            pltpu.make_async_copy(
                rows.at[pl.ds(0, CH)], out_hbm.at[pl.ds(0, CH)], osem
            ).wait()

        def chunk(m, carry):
            buf = lax.rem(m, NBUF)
            xbuf = lax.rem(m, 2)
            eb = base + m * CH

            pltpu.make_async_copy(
                x2_hbm.at[pl.ds(0, CH), :], xv.at[pl.ds(0, CH), :], xsem
            ).wait()

            @pl.when(m + 1 < NCH)
            def _():
                pltpu.async_copy(
                    x2_hbm.at[pl.ds(eb + CH, CH), :],
                    xv.at[pl.ds((1 - xbuf) * CH, CH), :],
                    xsem,
                )

            @pl.when(m >= NBUF)
            def _():
                wait_scatter()

            zero = lane * 0
            for t in range(CH // 16):
                rowi = lane + (xbuf * CH + t * 16)
                a = plsc.load_gather(xv, [rowi, zero])
                b = plsc.load_gather(xv, [rowi, zero + 1])
                c = plsc.load_gather(xv, [rowi, zero + 2])
                a = lax.min(lax.max(a, 0), V0 - 1)
                b = lax.min(lax.max(b, 0), V1 - 1)
                c = lax.min(lax.max(c, 0), V2 - 1)
                codev[pl.ds(buf * CH + t * 16, 16)] = (
                    a * (V1 * V2) + b * V2 + c
                )

            pltpu.async_copy(
                tcv.at[codev.at[pl.ds(buf * CH, CH)]],
                rows.at[pl.ds(buf * CH, CH)],
                gsem,
            )

            @pl.when(m >= 1)
            def _():
                wait_gather()
                prev = lax.rem(m - 1, NBUF)
                pltpu.async_copy(
                    rows.at[pl.ds(prev * CH, CH)],
                    out_hbm.at[pl.ds(eb - CH, CH)],
                    osem,
                )

            return carry

        lax.fori_loop(0, NCH, chunk, 0)

        wait_gather()
        lastbuf = (NCH - 1) % NBUF
        pltpu.async_copy(
            rows.at[pl.ds(lastbuf * CH, CH)],
            out_hbm.at[pl.ds(base + (NCH - 1) * CH, CH)],
            osem,
        )
        for _ in range(NBUF):
            wait_scatter()

    return lookup


def kernel(x, W0, W1, W2):
    E = x.shape[0]
    tcomb = _combine_tables(W0, W1, W2)
    return _make_sc_lookup(E)(tcomb, x.astype(jnp.int32))


# speedup vs baseline: 1.3458x; 1.2792x over previous
import functools

import jax
import jax.numpy as jnp
from jax import lax
from jax.experimental import pallas as pl
from jax.experimental.pallas import tpu as pltpu
from jax.experimental.pallas import tpu_sc as plsc

D = 128
V0, V1, V2 = 5, 6, 2
TROWS = 64


def _combine_body(w0_ref, w1_ref, w2_ref, out_ref):
    code = lax.broadcasted_iota(jnp.int32, (TROWS, 1), 0)
    i0 = code // (V1 * V2)
    i1 = (code // V2) % V1
    i2 = code % V2
    acc = jnp.zeros((TROWS, D), jnp.float32)
    for v in range(V0):
        acc = acc + jnp.where(i0 == v, w0_ref[v : v + 1, :], 0.0)
    for v in range(V1):
        acc = acc + jnp.where(i1 == v, w1_ref[v : v + 1, :], 0.0)
    for v in range(V2):
        acc = acc + jnp.where(i2 == v, w2_ref[v : v + 1, :], 0.0)
    out_ref[...] = acc


def _combine_tables(W0, W1, W2):
    return pl.pallas_call(
        _combine_body,
        out_shape=jax.ShapeDtypeStruct((TROWS, D), jnp.float32),
    )(W0, W1, W2)


def _make_sc_lookup(E):
    NW = 32
    per_w = E // NW
    CH = 80
    NCH = per_w // CH
    NBUF = 6
    assert per_w * NW == E and NCH * CH == per_w
    assert CH % 16 == 0 and CH <= 128 and CH % 8 == 0 and NCH > NBUF

    mesh = plsc.VectorSubcoreMesh(core_axis_name="c", subcore_axis_name="s")

    @functools.partial(
        pl.kernel,
        out_type=jax.ShapeDtypeStruct((E, D), jnp.float32),
        mesh=mesh,
        scratch_types=[
            pltpu.VMEM_SHARED((TROWS, D), jnp.float32),
            pltpu.VMEM((4 * CH, 3), jnp.int32),
            pltpu.VMEM((NBUF * CH,), jnp.int32),
            pltpu.VMEM((NBUF * CH, D), jnp.float32),
            pltpu.SemaphoreType.DMA,
            pltpu.SemaphoreType.DMA,
            pltpu.SemaphoreType.DMA,
        ],
        compiler_params=pltpu.CompilerParams(needs_layout_passes=False),
    )
    def lookup(tcomb_hbm, x2_hbm, out_hbm, tcv, xv, codev, rows, xsem, gsem, osem):
        cid = lax.axis_index("c")
        sid = lax.axis_index("s")
        wid = sid * 2 + cid
        base = wid * per_w
        lane = lax.iota(jnp.int32, 16)
        @pl.when(sid == 0)
        def _():
            pltpu.sync_copy(tcomb_hbm, tcv)

        plsc.subcore_barrier()
        for pf in range(3):
            pltpu.async_copy(
                x2_hbm.at[pl.ds(base + pf * CH, CH), :],
                xv.at[pl.ds(pf * CH, CH), :],
                xsem,
            )

        def wait_gather():
            pltpu.make_async_copy(
                tcv.at[codev.at[pl.ds(0, CH)]], rows.at[pl.ds(0, CH)], gsem
            ).wait()

        def wait_scatter():
            pltpu.make_async_copy(
                rows.at[pl.ds(0, CH)], out_hbm.at[pl.ds(0, CH)], osem
            ).wait()

        def chunk(m, carry):
            buf = lax.rem(m, NBUF)
            xbuf = lax.rem(m, 4)
            eb = base + m * CH

            pltpu.make_async_copy(
                x2_hbm.at[pl.ds(0, CH), :], xv.at[pl.ds(0, CH), :], xsem
            ).wait()

            @pl.when(m + 3 < NCH)
            def _():
                pltpu.async_copy(
                    x2_hbm.at[pl.ds(eb + 3 * CH, CH), :],
                    xv.at[pl.ds(lax.rem(m + 3, 4) * CH, CH), :],
                    xsem,
                )

            @pl.when(m >= NBUF)
            def _():
                wait_scatter()

            zero = lane * 0
            for t in range(CH // 16):
                rowi = lane + (xbuf * CH + t * 16)
                a = plsc.load_gather(xv, [rowi, zero])
                b = plsc.load_gather(xv, [rowi, zero + 1])
                c = plsc.load_gather(xv, [rowi, zero + 2])
                a = lax.min(lax.max(a, 0), V0 - 1)
                b = lax.min(lax.max(b, 0), V1 - 1)
                c = lax.min(lax.max(c, 0), V2 - 1)
                codev[pl.ds(buf * CH + t * 16, 16)] = (
                    a * (V1 * V2) + b * V2 + c
                )

            pltpu.async_copy(
                tcv.at[codev.at[pl.ds(buf * CH, CH)]],
                rows.at[pl.ds(buf * CH, CH)],
                gsem,
            )

            @pl.when(m >= 1)
            def _():
                wait_gather()
                prev = lax.rem(m - 1, NBUF)
                pltpu.async_copy(
                    rows.at[pl.ds(prev * CH, CH)],
                    out_hbm.at[pl.ds(eb - CH, CH)],
                    osem,
                )

            return carry

        lax.fori_loop(0, NCH, chunk, 0)

        wait_gather()
        lastbuf = (NCH - 1) % NBUF
        pltpu.async_copy(
            rows.at[pl.ds(lastbuf * CH, CH)],
            out_hbm.at[pl.ds(base + (NCH - 1) * CH, CH)],
            osem,
        )
        for _ in range(NBUF):
            wait_scatter()

    return lookup


def kernel(x, W0, W1, W2):
    E = x.shape[0]
    tcomb = _combine_tables(W0, W1, W2)
    return _make_sc_lookup(E)(tcomb, x.astype(jnp.int32))
